# Initial kernel scaffold; baseline (speedup 1.0000x reference)
import jax
import jax.numpy as jnp
from jax.experimental import pallas as pl


def kernel(h, edge_index, edge_attr, eps, W1, b1, g1, bt1, W2, b2, g2, bt2):
    raise NotImplementedError("write your pallas kernel here")


# speedup vs baseline: 603.4860x; 603.4860x over previous
import functools

import jax
import jax.numpy as jnp
from jax import lax
from jax.experimental import pallas as pl
from jax.experimental.pallas import tpu as pltpu
from jax.experimental.pallas import tpu_sc as plsc

B, N, E, D = 4, 10000, 64000, 256
ROWS = B * N
NWORK = 32
RSPAN = 312
LAST_SPAN = N - RSPAN * (NWORK - 1)
ACC_ROWS = 336
GROW = 330
FCHUNK = 512
NFCH = E // FCHUNK
RND = 64
SELCAP = 640


def _sc_body(h2d, srcA, dstA, ea2d, agg,
             acc, dstc, srcc, sdst, ssrc, seid,
             dsti, srci, eidi, hbuf, eabuf, sem0, sem1):
    c = lax.axis_index("c")
    s = lax.axis_index("s")
    wid = c * 16 + s
    lo = wid * RSPAN
    span = jnp.where(wid == NWORK - 1, N - lo, RSPAN)
    hi = lo + span
    zeros16 = jnp.zeros((16,), jnp.float32)
    iota16 = lax.iota(jnp.int32, 16)

    def drain(cur, b):
        for v in range(RND // 16):
            sl = pl.ds(v * 16, 16)
            dsti[sl] = sdst[sl]
            srci[sl] = ssrc[sl]
            eidi[sl] = seid[sl]
        cph = pltpu.async_copy(h2d.at[srci], hbuf, sem0)
        cpe = pltpu.async_copy(ea2d.at[eidi], eabuf, sem1)
        cph.wait()
        cpe.wait()
        for g in range(RND // 16):
            rows = g * 16 + iota16
            dr = dsti[pl.ds(g * 16, 16)]

            def ccol(j, carry, rows=rows, dr=dr):
                for t in range(8):
                    cb = jnp.full((16,), j * 8 + t, jnp.int32)
                    hv = plsc.load_gather(hbuf, [rows, cb])
                    ev = plsc.load_gather(eabuf, [rows, cb])
                    plsc.addupdate_scatter(acc, [dr, cb],
                                           jnp.maximum(hv + ev, 0.0))
                return carry

            lax.fori_loop(0, D // 8, ccol, 0)
        for v in range((SELCAP - RND) // 16):
            dn = pl.ds(v * 16, 16)
            sn = pl.ds(RND + v * 16, 16)
            sdst[dn] = sdst[sn]
            ssrc[dn] = ssrc[sn]
            seid[dn] = seid[sn]
        return cur - RND

    def per_batch(b, carry):
        def zrow(r, cz):
            for i in range(D // 16):
                acc[r, pl.ds(i * 16, 16)] = zeros16
            return cz

        lax.fori_loop(0, ACC_ROWS, zrow, 0)

        def per_chunk(k, cur):
            off = k * FCHUNK
            pltpu.sync_copy(dstA.at[b, pl.ds(off, FCHUNK)], dstc)
            pltpu.sync_copy(srcA.at[b, pl.ds(off, FCHUNK)], srcc)
            for i in range(FCHUNK // 16):
                sl = pl.ds(i * 16, 16)
                dv = dstc[sl]
                m = (dv >= lo) & (dv < hi)
                mi = m.astype(jnp.int32)
                cs = plsc.cumsum(mi)
                pos = jnp.where(m, cur + cs - mi, SELCAP - 1)
                plsc.store_scatter(sdst, [pos], dv - lo)
                plsc.store_scatter(ssrc, [pos], srcc[sl] + b * N)
                plsc.store_scatter(seid, [pos],
                                   (b * E + off + i * 16) + iota16)
                cur = cur + jnp.max(cs)
            return lax.while_loop(lambda cu: cu >= RND,
                                  lambda cu: drain(cu, b), cur)

        cur = lax.fori_loop(0, NFCH, per_chunk, jnp.int32(0))
        for v in range(RND // 16):
            sl = pl.ds(cur + v * 16, 16)
            sdst[sl] = jnp.full((16,), GROW, jnp.int32)
            ssrc[sl] = jnp.full((16,), b * N, jnp.int32)
            seid[sl] = jnp.full((16,), b * E, jnp.int32)
        cur = jnp.where(cur > 0, jnp.int32(RND), jnp.int32(0))
        lax.while_loop(lambda cu: cu >= RND, lambda cu: drain(cu, b), cur)

        o0 = b * N + lo

        @pl.when(wid < NWORK - 1)
        def _():
            pltpu.sync_copy(acc.at[pl.ds(0, RSPAN)], agg.at[pl.ds(o0, RSPAN)])

        @pl.when(wid == NWORK - 1)
        def _():
            pltpu.sync_copy(acc.at[pl.ds(0, LAST_SPAN)],
                            agg.at[pl.ds(o0, LAST_SPAN)])

        return carry

    lax.fori_loop(0, B, per_batch, 0)


_sc_scatter = functools.partial(
    pl.kernel,
    out_type=jax.ShapeDtypeStruct((ROWS, D), jnp.float32),
    mesh=plsc.VectorSubcoreMesh(core_axis_name="c", subcore_axis_name="s"),
    compiler_params=pltpu.CompilerParams(needs_layout_passes=False),
    scratch_types=[
        pltpu.VMEM((ACC_ROWS, D), jnp.float32),
        pltpu.VMEM((FCHUNK,), jnp.int32),
        pltpu.VMEM((FCHUNK,), jnp.int32),
        pltpu.VMEM((SELCAP,), jnp.int32),
        pltpu.VMEM((SELCAP,), jnp.int32),
        pltpu.VMEM((SELCAP,), jnp.int32),
        pltpu.VMEM((RND,), jnp.int32),
        pltpu.VMEM((RND,), jnp.int32),
        pltpu.VMEM((RND,), jnp.int32),
        pltpu.VMEM((RND, D), jnp.float32),
        pltpu.VMEM((RND, D), jnp.float32),
        pltpu.SemaphoreType.DMA,
        pltpu.SemaphoreType.DMA,
    ],
)(_sc_body)


BLK = 2000
NBLK = ROWS // BLK


def _mlp1_body(eps_ref, h_ref, agg_ref, w1_ref, b1_ref,
               y_ref, ss_ref, sq_ref, acc_s, acc_q):
    i = pl.program_id(0)

    @pl.when(i == 0)
    def _():
        acc_s[...] = jnp.zeros_like(acc_s)
        acc_q[...] = jnp.zeros_like(acc_q)

    scale = 1.0 + eps_ref[0]
    x = scale * h_ref[...] + agg_ref[...]
    y = jnp.dot(x, w1_ref[...], preferred_element_type=jnp.float32) + b1_ref[...]
    y_ref[...] = y
    acc_s[...] += y.reshape(BLK // 8, 8, D).sum(axis=0)
    acc_q[...] += (y * y).reshape(BLK // 8, 8, D).sum(axis=0)

    @pl.when(i == NBLK - 1)
    def _():
        ss_ref[...] = acc_s[...]
        sq_ref[...] = acc_q[...]


def _mlp2_body(y_ref, ss_ref, sq_ref, g1_ref, bt1_ref, w2_ref, b2_ref,
               w_ref, ss2_ref, sq2_ref, acc_s, acc_q):
    i = pl.program_id(0)

    @pl.when(i == 0)
    def _():
        acc_s[...] = jnp.zeros_like(acc_s)
        acc_q[...] = jnp.zeros_like(acc_q)

    mu = jnp.sum(ss_ref[...], axis=0, keepdims=True) * (1.0 / ROWS)
    ex2 = jnp.sum(sq_ref[...], axis=0, keepdims=True) * (1.0 / ROWS)
    inv = lax.rsqrt(ex2 - mu * mu + 1e-5)
    z = jnp.maximum(g1_ref[...] * (y_ref[...] - mu) * inv + bt1_ref[...], 0.0)
    w = jnp.dot(z, w2_ref[...], preferred_element_type=jnp.float32) + b2_ref[...]
    w_ref[...] = w
    acc_s[...] += w.reshape(BLK // 8, 8, D).sum(axis=0)
    acc_q[...] += (w * w).reshape(BLK // 8, 8, D).sum(axis=0)

    @pl.when(i == NBLK - 1)
    def _():
        ss2_ref[...] = acc_s[...]
        sq2_ref[...] = acc_q[...]


def _bn2_body(w_ref, ss_ref, sq_ref, g2_ref, bt2_ref, o_ref):
    mu = jnp.sum(ss_ref[...], axis=0, keepdims=True) * (1.0 / ROWS)
    ex2 = jnp.sum(sq_ref[...], axis=0, keepdims=True) * (1.0 / ROWS)
    inv = lax.rsqrt(ex2 - mu * mu + 1e-5)
    o_ref[...] = g2_ref[...] * (w_ref[...] - mu) * inv + bt2_ref[...]


def _row_spec(i):
    return pl.BlockSpec((BLK, D), lambda i: (i, 0))


_FULL = pl.BlockSpec((None, None), None)


def _mlp1(eps, h2d, agg, W1, b1):
    return pl.pallas_call(
        _mlp1_body,
        grid=(NBLK,),
        in_specs=[
            pl.BlockSpec(memory_space=pltpu.SMEM),
            pl.BlockSpec((BLK, D), lambda i: (i, 0)),
            pl.BlockSpec((BLK, D), lambda i: (i, 0)),
            pl.BlockSpec((D, D), lambda i: (0, 0)),
            pl.BlockSpec((1, D), lambda i: (0, 0)),
        ],
        out_specs=[
            pl.BlockSpec((BLK, D), lambda i: (i, 0)),
            pl.BlockSpec((8, D), lambda i: (0, 0)),
            pl.BlockSpec((8, D), lambda i: (0, 0)),
        ],
        out_shape=[
            jax.ShapeDtypeStruct((ROWS, D), jnp.float32),
            jax.ShapeDtypeStruct((8, D), jnp.float32),
            jax.ShapeDtypeStruct((8, D), jnp.float32),
        ],
        scratch_shapes=[
            pltpu.VMEM((8, D), jnp.float32),
            pltpu.VMEM((8, D), jnp.float32),
        ],
    )(eps, h2d, agg, W1, b1)


def _mlp2(y, ss1, sq1, g1, bt1, W2, b2):
    return pl.pallas_call(
        _mlp2_body,
        grid=(NBLK,),
        in_specs=[
            pl.BlockSpec((BLK, D), lambda i: (i, 0)),
            pl.BlockSpec((8, D), lambda i: (0, 0)),
            pl.BlockSpec((8, D), lambda i: (0, 0)),
            pl.BlockSpec((1, D), lambda i: (0, 0)),
            pl.BlockSpec((1, D), lambda i: (0, 0)),
            pl.BlockSpec((D, D), lambda i: (0, 0)),
            pl.BlockSpec((1, D), lambda i: (0, 0)),
        ],
        out_specs=[
            pl.BlockSpec((BLK, D), lambda i: (i, 0)),
            pl.BlockSpec((8, D), lambda i: (0, 0)),
            pl.BlockSpec((8, D), lambda i: (0, 0)),
        ],
        out_shape=[
            jax.ShapeDtypeStruct((ROWS, D), jnp.float32),
            jax.ShapeDtypeStruct((8, D), jnp.float32),
            jax.ShapeDtypeStruct((8, D), jnp.float32),
        ],
        scratch_shapes=[
            pltpu.VMEM((8, D), jnp.float32),
            pltpu.VMEM((8, D), jnp.float32),
        ],
    )(y, ss1, sq1, g1, bt1, W2, b2)


def _bn2(w, ss2, sq2, g2, bt2):
    return pl.pallas_call(
        _bn2_body,
        grid=(NBLK,),
        in_specs=[
            pl.BlockSpec((BLK, D), lambda i: (i, 0)),
            pl.BlockSpec((8, D), lambda i: (0, 0)),
            pl.BlockSpec((8, D), lambda i: (0, 0)),
            pl.BlockSpec((1, D), lambda i: (0, 0)),
            pl.BlockSpec((1, D), lambda i: (0, 0)),
        ],
        out_specs=pl.BlockSpec((BLK, D), lambda i: (i, 0)),
        out_shape=jax.ShapeDtypeStruct((ROWS, D), jnp.float32),
    )(w, ss2, sq2, g2, bt2)


def kernel(h, edge_index, edge_attr, eps, W1, b1, g1, bt1, W2, b2, g2, bt2):
    h2d = h.reshape(ROWS, D)
    ea2d = edge_attr.reshape(B * E, D)
    src = edge_index[:, 0, :]
    dst = edge_index[:, 1, :]
    agg = _sc_scatter(h2d, src, dst, ea2d)
    y, ss1, sq1 = _mlp1(eps, h2d, agg, W1, b1.reshape(1, D))
    w, ss2, sq2 = _mlp2(y, ss1, sq1, g1.reshape(1, D), bt1.reshape(1, D),
                        W2, b2.reshape(1, D))
    out = _bn2(w, ss2, sq2, g2.reshape(1, D), bt2.reshape(1, D))
    return out.reshape(B, N, D)


# speedup vs baseline: 1150.1934x; 1.9059x over previous
import functools

import jax
import jax.numpy as jnp
from jax import lax
from jax.experimental import pallas as pl
from jax.experimental.pallas import tpu as pltpu
from jax.experimental.pallas import tpu_sc as plsc

B, N, E, D = 4, 10000, 64000, 256
ROWS = B * N
NWORK = 32
RSPAN = 312
LAST_SPAN = N - RSPAN * (NWORK - 1)
ACC_ROWS = 336
GROW = 330
FCHUNK = 1280
NFCH = E // FCHUNK
NPAIR = NFCH // 2
RND = 64
SELCAP = 1424
TRASH = SELCAP - 16


def _sc_body(h2d, srcA, dstA, ea2d, agg,
             acc, dstc0, srcc0, dstc1, srcc1, sdst, ssrc, seid,
             dsti, srci, eidi, hbuf, eabuf,
             semh, seme, semd0, sems0, semd1, sems1):
    c = lax.axis_index("c")
    s = lax.axis_index("s")
    wid = c * 16 + s
    lo = wid * RSPAN
    span = jnp.where(wid == NWORK - 1, N - lo, RSPAN)
    hi = lo + span
    zeros16 = jnp.zeros((16,), jnp.float32)
    iota16 = lax.iota(jnp.int32, 16)

    def drain(cur, b):
        for v in range(RND // 16):
            sl = pl.ds(v * 16, 16)
            dsti[sl] = sdst[sl]
            srci[sl] = ssrc[sl]
            eidi[sl] = seid[sl]
        cph = pltpu.async_copy(h2d.at[srci], hbuf, semh)
        cpe = pltpu.async_copy(ea2d.at[eidi], eabuf, seme)
        cph.wait()
        cpe.wait()

        def edge_grp(v, cy):
            dr = dsti[pl.ds(v * 16, 16)]
            for i in range(16):
                r = dr[i]
                e = v * 16 + i

                def colgrp(j, cz, r=r, e=e):
                    for t in range(4):
                        slc = pl.ds(j * 64 + t * 16, 16)
                        acc[r, slc] = acc[r, slc] + jnp.maximum(
                            hbuf[e, slc] + eabuf[e, slc], 0.0)
                    return cz

                lax.fori_loop(0, D // 64, colgrp, 0)
            return cy

        lax.fori_loop(0, RND // 16, edge_grp, 0)

        nshift = (cur - RND + 31) // 16

        def shift(v, cz):
            dn = pl.ds(v * 16, 16)
            sn = pl.ds(RND + v * 16, 16)
            sdst[dn] = sdst[sn]
            ssrc[dn] = ssrc[sn]
            seid[dn] = seid[sn]
            return cz

        lax.fori_loop(0, nshift, shift, 0)
        return cur - RND

    def filt(dstcb, srccb, b, off, cur):
        cv0 = jnp.full((16,), cur, jnp.int32)

        def grp(g, cv):
            for t in range(4):
                i0 = g * 4 + t
                sl = pl.ds(i0 * 16, 16)
                dv = dstcb[sl]
                m = (dv >= lo) & (dv < hi)
                mi = m.astype(jnp.int32)
                cs = plsc.cumsum(mi)
                pos = jnp.where(m, cv + (cs - mi), TRASH + iota16)
                plsc.store_scatter(sdst, [pos], dv - lo)
                plsc.store_scatter(ssrc, [pos], srccb[sl] + b * N)
                plsc.store_scatter(seid, [pos],
                                   (b * E + off + i0 * 16) + iota16)
                cv = cv + plsc.all_reduce_population_count(m)
            return cv

        cv = lax.fori_loop(0, FCHUNK // 64, grp, cv0)
        return jnp.max(cv)

    def per_batch(b, carry):
        def zrow(r, cz):
            for i in range(D // 16):
                acc[r, pl.ds(i * 16, 16)] = zeros16
            return cz

        lax.fori_loop(0, ACC_ROWS, zrow, 0)

        pltpu.async_copy(dstA.at[b, pl.ds(0, FCHUNK)], dstc0, semd0)
        pltpu.async_copy(srcA.at[b, pl.ds(0, FCHUNK)], srcc0, sems0)

        def pair(k2, cur):
            off0 = (2 * k2) * FCHUNK
            off1 = off0 + FCHUNK
            off2 = jnp.minimum(off1 + FCHUNK, (NFCH - 1) * FCHUNK)
            pltpu.make_async_copy(dstA.at[b, pl.ds(0, FCHUNK)],
                                  dstc0, semd0).wait()
            pltpu.make_async_copy(srcA.at[b, pl.ds(0, FCHUNK)],
                                  srcc0, sems0).wait()
            pltpu.async_copy(dstA.at[b, pl.ds(off1, FCHUNK)], dstc1, semd1)
            pltpu.async_copy(srcA.at[b, pl.ds(off1, FCHUNK)], srcc1, sems1)
            cur = filt(dstc0, srcc0, b, off0, cur)
            cur = lax.while_loop(lambda cu: cu >= RND,
                                 lambda cu: drain(cu, b), cur)
            pltpu.make_async_copy(dstA.at[b, pl.ds(0, FCHUNK)],
                                  dstc1, semd1).wait()
            pltpu.make_async_copy(srcA.at[b, pl.ds(0, FCHUNK)],
                                  srcc1, sems1).wait()
            pltpu.async_copy(dstA.at[b, pl.ds(off2, FCHUNK)], dstc0, semd0)
            pltpu.async_copy(srcA.at[b, pl.ds(off2, FCHUNK)], srcc0, sems0)
            cur = filt(dstc1, srcc1, b, off1, cur)
            cur = lax.while_loop(lambda cu: cu >= RND,
                                 lambda cu: drain(cu, b), cur)
            return cur

        cur = lax.fori_loop(0, NPAIR, pair, jnp.int32(0))
        pltpu.make_async_copy(dstA.at[b, pl.ds(0, FCHUNK)],
                              dstc0, semd0).wait()
        pltpu.make_async_copy(srcA.at[b, pl.ds(0, FCHUNK)],
                              srcc0, sems0).wait()

        for v in range(RND // 16):
            sl = pl.ds(cur + v * 16, 16)
            sdst[sl] = jnp.full((16,), GROW, jnp.int32)
            ssrc[sl] = jnp.full((16,), b * N, jnp.int32)
            seid[sl] = jnp.full((16,), b * E, jnp.int32)
        cur = jnp.where(cur > 0, jnp.int32(RND), jnp.int32(0))
        lax.while_loop(lambda cu: cu >= RND, lambda cu: drain(cu, b), cur)

        o0 = b * N + lo

        @pl.when(wid < NWORK - 1)
        def _():
            pltpu.sync_copy(acc.at[pl.ds(0, RSPAN)], agg.at[pl.ds(o0, RSPAN)])

        @pl.when(wid == NWORK - 1)
        def _():
            pltpu.sync_copy(acc.at[pl.ds(0, LAST_SPAN)],
                            agg.at[pl.ds(o0, LAST_SPAN)])

        return carry

    lax.fori_loop(0, B, per_batch, 0)


_sc_scatter = functools.partial(
    pl.kernel,
    out_type=jax.ShapeDtypeStruct((ROWS, D), jnp.float32),
    mesh=plsc.VectorSubcoreMesh(core_axis_name="c", subcore_axis_name="s"),
    compiler_params=pltpu.CompilerParams(needs_layout_passes=False),
    scratch_types=[
        pltpu.VMEM((ACC_ROWS, D), jnp.float32),
        pltpu.VMEM((FCHUNK,), jnp.int32),
        pltpu.VMEM((FCHUNK,), jnp.int32),
        pltpu.VMEM((FCHUNK,), jnp.int32),
        pltpu.VMEM((FCHUNK,), jnp.int32),
        pltpu.VMEM((SELCAP,), jnp.int32),
        pltpu.VMEM((SELCAP,), jnp.int32),
        pltpu.VMEM((SELCAP,), jnp.int32),
        pltpu.VMEM((RND,), jnp.int32),
        pltpu.VMEM((RND,), jnp.int32),
        pltpu.VMEM((RND,), jnp.int32),
        pltpu.VMEM((RND, D), jnp.float32),
        pltpu.VMEM((RND, D), jnp.float32),
        pltpu.SemaphoreType.DMA,
        pltpu.SemaphoreType.DMA,
        pltpu.SemaphoreType.DMA,
        pltpu.SemaphoreType.DMA,
        pltpu.SemaphoreType.DMA,
        pltpu.SemaphoreType.DMA,
    ],
)(_sc_body)


BLK = 2000
NBLK = ROWS // BLK


def _mlp1_body(eps_ref, h_ref, agg_ref, w1_ref, b1_ref,
               y_ref, ss_ref, sq_ref, acc_s, acc_q):
    i = pl.program_id(0)

    @pl.when(i == 0)
    def _():
        acc_s[...] = jnp.zeros_like(acc_s)
        acc_q[...] = jnp.zeros_like(acc_q)

    scale = 1.0 + eps_ref[0]
    x = scale * h_ref[...] + agg_ref[...]
    y = jnp.dot(x, w1_ref[...], preferred_element_type=jnp.float32) + b1_ref[...]
    y_ref[...] = y
    acc_s[...] += y.reshape(BLK // 8, 8, D).sum(axis=0)
    acc_q[...] += (y * y).reshape(BLK // 8, 8, D).sum(axis=0)

    @pl.when(i == NBLK - 1)
    def _():
        ss_ref[...] = acc_s[...]
        sq_ref[...] = acc_q[...]


def _mlp2_body(y_ref, ss_ref, sq_ref, g1_ref, bt1_ref, w2_ref, b2_ref,
               w_ref, ss2_ref, sq2_ref, acc_s, acc_q):
    i = pl.program_id(0)

    @pl.when(i == 0)
    def _():
        acc_s[...] = jnp.zeros_like(acc_s)
        acc_q[...] = jnp.zeros_like(acc_q)

    mu = jnp.sum(ss_ref[...], axis=0, keepdims=True) * (1.0 / ROWS)
    ex2 = jnp.sum(sq_ref[...], axis=0, keepdims=True) * (1.0 / ROWS)
    inv = lax.rsqrt(ex2 - mu * mu + 1e-5)
    z = jnp.maximum(g1_ref[...] * (y_ref[...] - mu) * inv + bt1_ref[...], 0.0)
    w = jnp.dot(z, w2_ref[...], preferred_element_type=jnp.float32) + b2_ref[...]
    w_ref[...] = w
    acc_s[...] += w.reshape(BLK // 8, 8, D).sum(axis=0)
    acc_q[...] += (w * w).reshape(BLK // 8, 8, D).sum(axis=0)

    @pl.when(i == NBLK - 1)
    def _():
        ss2_ref[...] = acc_s[...]
        sq2_ref[...] = acc_q[...]


def _bn2_body(w_ref, ss_ref, sq_ref, g2_ref, bt2_ref, o_ref):
    mu = jnp.sum(ss_ref[...], axis=0, keepdims=True) * (1.0 / ROWS)
    ex2 = jnp.sum(sq_ref[...], axis=0, keepdims=True) * (1.0 / ROWS)
    inv = lax.rsqrt(ex2 - mu * mu + 1e-5)
    o_ref[...] = g2_ref[...] * (w_ref[...] - mu) * inv + bt2_ref[...]


def _row_spec(i):
    return pl.BlockSpec((BLK, D), lambda i: (i, 0))


_FULL = pl.BlockSpec((None, None), None)


def _mlp1(eps, h2d, agg, W1, b1):
    return pl.pallas_call(
        _mlp1_body,
        grid=(NBLK,),
        in_specs=[
            pl.BlockSpec(memory_space=pltpu.SMEM),
            pl.BlockSpec((BLK, D), lambda i: (i, 0)),
            pl.BlockSpec((BLK, D), lambda i: (i, 0)),
            pl.BlockSpec((D, D), lambda i: (0, 0)),
            pl.BlockSpec((1, D), lambda i: (0, 0)),
        ],
        out_specs=[
            pl.BlockSpec((BLK, D), lambda i: (i, 0)),
            pl.BlockSpec((8, D), lambda i: (0, 0)),
            pl.BlockSpec((8, D), lambda i: (0, 0)),
        ],
        out_shape=[
            jax.ShapeDtypeStruct((ROWS, D), jnp.float32),
            jax.ShapeDtypeStruct((8, D), jnp.float32),
            jax.ShapeDtypeStruct((8, D), jnp.float32),
        ],
        scratch_shapes=[
            pltpu.VMEM((8, D), jnp.float32),
            pltpu.VMEM((8, D), jnp.float32),
        ],
    )(eps, h2d, agg, W1, b1)


def _mlp2(y, ss1, sq1, g1, bt1, W2, b2):
    return pl.pallas_call(
        _mlp2_body,
        grid=(NBLK,),
        in_specs=[
            pl.BlockSpec((BLK, D), lambda i: (i, 0)),
            pl.BlockSpec((8, D), lambda i: (0, 0)),
            pl.BlockSpec((8, D), lambda i: (0, 0)),
            pl.BlockSpec((1, D), lambda i: (0, 0)),
            pl.BlockSpec((1, D), lambda i: (0, 0)),
            pl.BlockSpec((D, D), lambda i: (0, 0)),
            pl.BlockSpec((1, D), lambda i: (0, 0)),
        ],
        out_specs=[
            pl.BlockSpec((BLK, D), lambda i: (i, 0)),
            pl.BlockSpec((8, D), lambda i: (0, 0)),
            pl.BlockSpec((8, D), lambda i: (0, 0)),
        ],
        out_shape=[
            jax.ShapeDtypeStruct((ROWS, D), jnp.float32),
            jax.ShapeDtypeStruct((8, D), jnp.float32),
            jax.ShapeDtypeStruct((8, D), jnp.float32),
        ],
        scratch_shapes=[
            pltpu.VMEM((8, D), jnp.float32),
            pltpu.VMEM((8, D), jnp.float32),
        ],
    )(y, ss1, sq1, g1, bt1, W2, b2)


def _bn2(w, ss2, sq2, g2, bt2):
    return pl.pallas_call(
        _bn2_body,
        grid=(NBLK,),
        in_specs=[
            pl.BlockSpec((BLK, D), lambda i: (i, 0)),
            pl.BlockSpec((8, D), lambda i: (0, 0)),
            pl.BlockSpec((8, D), lambda i: (0, 0)),
            pl.BlockSpec((1, D), lambda i: (0, 0)),
            pl.BlockSpec((1, D), lambda i: (0, 0)),
        ],
        out_specs=pl.BlockSpec((BLK, D), lambda i: (i, 0)),
        out_shape=jax.ShapeDtypeStruct((ROWS, D), jnp.float32),
    )(w, ss2, sq2, g2, bt2)


def kernel(h, edge_index, edge_attr, eps, W1, b1, g1, bt1, W2, b2, g2, bt2):
    h2d = h.reshape(ROWS, D)
    ea2d = edge_attr.reshape(B * E, D)
    src = edge_index[:, 0, :]
    dst = edge_index[:, 1, :]
    agg = _sc_scatter(h2d, src, dst, ea2d)
    y, ss1, sq1 = _mlp1(eps, h2d, agg, W1, b1.reshape(1, D))
    w, ss2, sq2 = _mlp2(y, ss1, sq1, g1.reshape(1, D), bt1.reshape(1, D),
                        W2, b2.reshape(1, D))
    out = _bn2(w, ss2, sq2, g2.reshape(1, D), bt2.reshape(1, D))
    return out.reshape(B, N, D)
